# baseline (speedup 1.0000x reference)
import functools

import jax
import jax.numpy as jnp
from jax import lax
from jax.experimental import pallas as pl
from jax.experimental.pallas import tpu as pltpu
from jax.experimental.pallas import tpu_sc as plsc

N_TABLES = 26
VOCAB = 100000
EMB = 32
HID = 64
NL = 2
B = 16384

R = B * N_TABLES
NC, NS = 2, 16
NW = NC * NS

VC = 768
WPT = 130
VTAIL = 256
UA = N_TABLES * WPT
UA_LO = UA // NW
UA_EXTRA = UA - UA_LO * NW
QROWS = VC * EMB // 128
QTAIL = VTAIL * EMB // 128

NCB = (N_TABLES * EMB + 127) // 128
ER = (B // 8) * NCB * 32
RPW = R // NW
SPW = ER // NW
G = 128
NGPW = SPW // G
SCG = 8
CHUNK = G * SCG
NCH = NGPW // SCG
TPG = RPW // G


@functools.partial(
    pl.kernel,
    out_type=jax.ShapeDtypeStruct((N_TABLES * VOCAB * EMB,), jnp.float32),
    mesh=plsc.VectorSubcoreMesh(core_axis_name="c", subcore_axis_name="s"),
    compiler_params=pltpu.CompilerParams(needs_layout_passes=False),
    scratch_types=[
        pltpu.VMEM((EMB, VC), jnp.float32),
        pltpu.VMEM((EMB, VC), jnp.float32),
        pltpu.VMEM((EMB * (VC + 1),), jnp.float32),
        pltpu.VMEM((VC * EMB,), jnp.float32),
        pltpu.VMEM((VC * EMB,), jnp.float32),
        pltpu.SemaphoreType.DMA,
        pltpu.SemaphoreType.DMA,
        pltpu.SemaphoreType.DMA,
        pltpu.SemaphoreType.DMA,
    ],
)
def _sc_repack(tab_hbm, tail_hbm, packed_hbm, buf0, buf1, pb, ob0, ob1,
               rd0, rd1, wr0, wr1):
    wid = lax.axis_index("s") * NC + lax.axis_index("c")
    u0 = wid * UA_LO + jnp.minimum(wid, UA_EXTRA)
    cnt = UA_LO + jnp.where(wid < UA_EXTRA, 1, 0)
    u1 = u0 + cnt

    bufs = (buf0, buf1)
    obs = (ob0, ob1)
    rds = (rd0, rd1)
    wrs = (wr0, wr1)
    lanes = lax.iota(jnp.int32, 16)
    OUTW = VC * EMB

    def unit_nv(u):
        n = u // WPT
        v0 = pl.multiple_of((u % WPT) * VC, 128)
        return n, v0

    PITCH = VC + 1

    def fire_read(u, p):
        n, v0 = unit_nv(u)
        pltpu.async_copy(tab_hbm.at[n, :, pl.ds(v0, VC)], bufs[p], rds[p])

    fire_read(u0, 0)
    fire_read(u0 + 1, 1)

    def transpose_window(buf, ob, nv):
        @plsc.parallel_loop(0, nv // 16, unroll=2)
        def _repitch(s):
            v = s * 16
            for d in range(EMB):
                pb[pl.ds(d * PITCH + v, 16)] = buf[d, pl.ds(v, 16)]

        idx0 = lanes * PITCH
        idx_hi = idx0 + 16 * PITCH

        @plsc.parallel_loop(0, nv, unroll=4,
                            carry=(idx0, idx_hi, jnp.int32(0)))
        def _steps(v, c):
            ia, ib, off = c
            g0 = plsc.load_gather(pb, [ia])
            g1 = plsc.load_gather(pb, [ib])
            ob[pl.ds(off, 16)] = g0
            ob[pl.ds(off + 16, 16)] = g1
            return ia + 1, ib + 1, off + EMB

    def loop_body(i2, carry):
        for p in range(2):
            u = u0 + 2 * i2 + p

            @pl.when(u < u1)
            def _do():
                n, v0 = unit_nv(u)
                pltpu.make_async_copy(
                    tab_hbm.at[0, :, pl.ds(0, VC)], bufs[p], rds[p]
                ).wait()

                @pl.when(u >= u0 + 2)
                def _wait_prev_write():
                    pltpu.make_async_copy(
                        obs[p], packed_hbm.at[pl.ds(0, OUTW)], wrs[p]
                    ).wait()

                transpose_window(bufs[p], obs[p], VC)
                flat0 = pl.multiple_of(n * (VOCAB * EMB) + v0 * EMB, 1024)
                pltpu.async_copy(
                    obs[p], packed_hbm.at[pl.ds(flat0, OUTW)], wrs[p]
                )

                @pl.when(u + 2 < u1)
                def _next_read():
                    fire_read(u + 2, p)

        return carry

    lax.fori_loop(0, (cnt + 1) // 2, loop_body, 0)

    for p in range(2):
        pltpu.make_async_copy(
            obs[p], packed_hbm.at[pl.ds(0, OUTW)], wrs[p]
        ).wait()

    @pl.when(wid < N_TABLES)
    def _tail():
        n = wid
        pltpu.sync_copy(tail_hbm.at[n], buf0.at[:, pl.ds(0, VTAIL)])
        transpose_window(buf0, ob0, VTAIL)
        flat0 = pl.multiple_of(
            n * (VOCAB * EMB) + (VOCAB - VTAIL) * EMB, 1024
        )
        pltpu.sync_copy(ob0.at[pl.ds(0, VTAIL * EMB)],
                        packed_hbm.at[pl.ds(flat0, VTAIL * EMB)])


@functools.partial(
    pl.kernel,
    out_type=jax.ShapeDtypeStruct((ER, EMB), jnp.float32),
    mesh=plsc.VectorSubcoreMesh(core_axis_name="c", subcore_axis_name="s"),
    compiler_params=pltpu.CompilerParams(
        use_tc_tiling_on_sc=False, needs_layout_passes=False
    ),
    scratch_types=[
        pltpu.VMEM((TPG, G), jnp.int32),
        pltpu.VMEM((NGPW, G), jnp.int32),
        pltpu.VMEM((CHUNK, EMB), jnp.float32),
        pltpu.SemaphoreType.DMA,
    ],
)
def _sc_gather(tokens_hbm, table_hbm, out_hbm, tok_v, idx_v, buf, sem):
    wid = lax.axis_index("s") * NC + lax.axis_index("c")

    pltpu.sync_copy(tokens_hbm.at[pl.ds(wid * TPG, TPG), :], tok_v)

    lanes = lax.iota(jnp.int32, 16)
    br_l = lanes >> 2
    q_l = lanes & 3

    def idx_body(j, carry):
        for s in range(G // 16):
            off = j * G + s * 16
            blk = off >> 5
            half = (off & 31) >> 4
            bb = blk // NCB
            cb = blk % NCB
            b_loc = bb * 8 + half * 4 + br_l
            n = cb * 4 + q_l
            tf = b_loc * N_TABLES + n
            tok = plsc.load_gather(tok_v, [tf >> 7, tf & (G - 1)])
            pad = n >= N_TABLES
            idx = jnp.where(pad, 0, tok + n * VOCAB)
            idx_v[j, pl.ds(s * 16, 16)] = idx
        return carry

    lax.fori_loop(0, NGPW, idx_body, 0)
    base = wid * SPW

    def chunk_body(c, carry):
        copies = []
        for g in range(SCG):
            copies.append(
                pltpu.async_copy(
                    table_hbm.at[idx_v.at[c * SCG + g]],
                    buf.at[pl.ds(g * G, G), :],
                    sem,
                )
            )
        for cp in copies:
            cp.wait()
        pltpu.sync_copy(buf, out_hbm.at[pl.ds(base + c * CHUNK, CHUNK), :])
        return carry

    lax.fori_loop(0, NCH, chunk_body, 0)


BLK = 2048
LASTK = N_TABLES * EMB - (NCB - 1) * 128


def _mm_body(e_ref, m_ref, c_ref, o_ref):
    acc = c_ref[...] + jnp.zeros((BLK, NL * HID), jnp.float32)
    for cb in range(NCB):
        x = e_ref[:, cb].reshape(BLK, 128)
        if cb == NCB - 1:
            acc += jnp.dot(x[:, :LASTK], m_ref[cb, :LASTK],
                           preferred_element_type=jnp.float32)
        else:
            acc += jnp.dot(x, m_ref[cb],
                           preferred_element_type=jnp.float32)
    acc_t = acc.T
    o_ref[0] = acc_t[:HID]
    o_ref[1] = acc_t[HID:]


_mm = pl.pallas_call(
    _mm_body,
    grid=(B // BLK,),
    in_specs=[
        pl.BlockSpec((BLK // 8, NCB, 8, 128), lambda i: (i, 0, 0, 0)),
        pl.BlockSpec((NCB, 128, NL * HID), lambda i: (0, 0, 0)),
        pl.BlockSpec((1, NL * HID), lambda i: (0, 0)),
    ],
    out_specs=pl.BlockSpec((NL, HID, BLK), lambda i: (0, 0, i)),
    out_shape=jax.ShapeDtypeStruct((NL, HID, B), jnp.float32),
)


def kernel(tokens, tables, W_embed_lin, b_embed_lin, W_final, b_final):
    tokens_flat = tokens.astype(jnp.int32).reshape(R // G, G)

    tables_t = jnp.transpose(tables, (0, 2, 1))
    tail = tables_t[:, :, VOCAB - 256:]
    packed = _sc_repack(tables_t, tail)
    table_flat = packed.reshape(N_TABLES * VOCAB, EMB)

    M = jnp.einsum("ln,nhd->ndlh", W_final, W_embed_lin).reshape(
        N_TABLES * EMB, NL * HID
    )
    Mp = jnp.concatenate(
        [M.reshape(N_TABLES, EMB, NL * HID),
         jnp.zeros((4 * NCB - N_TABLES, EMB, NL * HID), jnp.float32)]
    ).reshape(NCB, 128, NL * HID)
    const = (W_final @ b_embed_lin + b_final[:, None]).reshape(1, NL * HID)

    rows = _sc_gather(tokens_flat, table_flat)
    E4 = rows.reshape(B // 8, NCB, 8, 128)
    out_t = _mm(E4, Mp, const)
    return jnp.transpose(out_t, (0, 2, 1))


# speedup vs baseline: 2.1255x; 2.1255x over previous
import functools

import jax
import jax.numpy as jnp
from jax import lax
from jax.experimental import pallas as pl
from jax.experimental.pallas import tpu as pltpu
from jax.experimental.pallas import tpu_sc as plsc

N_TABLES = 26
VOCAB = 100000
EMB = 32
HID = 64
NL = 2
B = 16384

R = B * N_TABLES
NC, NS = 2, 16
NW = NC * NS

VC = 768
WPT = 130
VTAIL = 256
UA = N_TABLES * WPT
UA_LO = UA // NW
UA_EXTRA = UA - UA_LO * NW
QROWS = VC * EMB // 128
QTAIL = VTAIL * EMB // 128

NCB = (N_TABLES * EMB + 127) // 128
ER = (B // 8) * NCB * 32
RPW = R // NW
SPW = ER // NW
G = 128
NGPW = SPW // G
SCG = 8
CHUNK = G * SCG
NCH = NGPW // SCG
TPG = RPW // G


@functools.partial(
    pl.kernel,
    out_type=jax.ShapeDtypeStruct((N_TABLES * VOCAB * EMB,), jnp.float32),
    mesh=plsc.VectorSubcoreMesh(core_axis_name="c", subcore_axis_name="s"),
    compiler_params=pltpu.CompilerParams(needs_layout_passes=False),
    scratch_types=[
        pltpu.VMEM((EMB, VC), jnp.float32),
        pltpu.VMEM((EMB, VC), jnp.float32),
        pltpu.VMEM((EMB * (VC + 1),), jnp.float32),
        pltpu.VMEM((VC * EMB,), jnp.float32),
        pltpu.VMEM((VC * EMB,), jnp.float32),
        pltpu.SemaphoreType.DMA,
        pltpu.SemaphoreType.DMA,
        pltpu.SemaphoreType.DMA,
        pltpu.SemaphoreType.DMA,
    ],
)
def _sc_repack(tab_hbm, tail_hbm, packed_hbm, buf0, buf1, pb, ob0, ob1,
               rd0, rd1, wr0, wr1):
    wid = lax.axis_index("s") * NC + lax.axis_index("c")
    u0 = wid * UA_LO + jnp.minimum(wid, UA_EXTRA)
    cnt = UA_LO + jnp.where(wid < UA_EXTRA, 1, 0)
    u1 = u0 + cnt

    bufs = (buf0, buf1)
    obs = (ob0, ob1)
    rds = (rd0, rd1)
    wrs = (wr0, wr1)
    lanes = lax.iota(jnp.int32, 16)
    OUTW = VC * EMB

    def unit_nv(u):
        n = u // WPT
        v0 = pl.multiple_of((u % WPT) * VC, 128)
        return n, v0

    PITCH = VC + 1

    def fire_read(u, p):
        n, v0 = unit_nv(u)
        pltpu.async_copy(tab_hbm.at[n, :, pl.ds(v0, VC)], bufs[p], rds[p])

    fire_read(u0, 0)
    fire_read(u0 + 1, 1)

    def transpose_window(buf, ob, nv):
        @plsc.parallel_loop(0, nv // 16, unroll=2)
        def _repitch(s):
            v = s * 16
            for d in range(EMB):
                pb[pl.ds(d * PITCH + v, 16)] = buf[d, pl.ds(v, 16)]

        idx0 = lanes * PITCH
        idx_hi = idx0 + 16 * PITCH

        @plsc.parallel_loop(0, nv, unroll=4,
                            carry=(idx0, idx_hi, jnp.int32(0)))
        def _steps(v, c):
            ia, ib, off = c
            g0 = plsc.load_gather(pb, [ia])
            g1 = plsc.load_gather(pb, [ib])
            ob[pl.ds(off, 16)] = g0
            ob[pl.ds(off + 16, 16)] = g1
            return ia + 1, ib + 1, off + EMB

    def loop_body(i2, carry):
        for p in range(2):
            u = u0 + 2 * i2 + p

            @pl.when(u < u1)
            def _do():
                n, v0 = unit_nv(u)
                pltpu.make_async_copy(
                    tab_hbm.at[0, :, pl.ds(0, VC)], bufs[p], rds[p]
                ).wait()

                @pl.when(u >= u0 + 2)
                def _wait_prev_write():
                    pltpu.make_async_copy(
                        obs[p], packed_hbm.at[pl.ds(0, OUTW)], wrs[p]
                    ).wait()

                transpose_window(bufs[p], obs[p], VC)
                flat0 = pl.multiple_of(n * (VOCAB * EMB) + v0 * EMB, 1024)
                pltpu.async_copy(
                    obs[p], packed_hbm.at[pl.ds(flat0, OUTW)], wrs[p]
                )

                @pl.when(u + 2 < u1)
                def _next_read():
                    fire_read(u + 2, p)

        return carry

    lax.fori_loop(0, (cnt + 1) // 2, loop_body, 0)

    for p in range(2):
        pltpu.make_async_copy(
            obs[p], packed_hbm.at[pl.ds(0, OUTW)], wrs[p]
        ).wait()

    @pl.when(wid < N_TABLES)
    def _tail():
        n = wid
        pltpu.sync_copy(tail_hbm.at[n], buf0.at[:, pl.ds(0, VTAIL)])
        transpose_window(buf0, ob0, VTAIL)
        flat0 = pl.multiple_of(
            n * (VOCAB * EMB) + (VOCAB - VTAIL) * EMB, 1024
        )
        pltpu.sync_copy(ob0.at[pl.ds(0, VTAIL * EMB)],
                        packed_hbm.at[pl.ds(flat0, VTAIL * EMB)])


@functools.partial(
    pl.kernel,
    out_type=jax.ShapeDtypeStruct((ER, EMB), jnp.float32),
    mesh=plsc.VectorSubcoreMesh(core_axis_name="c", subcore_axis_name="s"),
    compiler_params=pltpu.CompilerParams(
        use_tc_tiling_on_sc=False, needs_layout_passes=False
    ),
    scratch_types=[
        pltpu.VMEM((TPG, G), jnp.int32),
        pltpu.VMEM((NGPW, G), jnp.int32),
        pltpu.VMEM((CHUNK, EMB), jnp.float32),
        pltpu.SemaphoreType.DMA,
    ],
)
def _sc_gather(tokens_hbm, table_hbm, out_hbm, tok_v, idx_v, buf, sem):
    wid = lax.axis_index("s") * NC + lax.axis_index("c")

    pltpu.sync_copy(tokens_hbm.at[pl.ds(wid * TPG, TPG), :], tok_v)

    lanes = lax.iota(jnp.int32, 16)
    br_l = lanes >> 2
    q_l = lanes & 3

    def idx_body(j, carry):
        for s in range(G // 16):
            off = j * G + s * 16
            blk = off >> 5
            half = (off & 31) >> 4
            bb = blk // NCB
            cb = blk % NCB
            b_loc = bb * 8 + half * 4 + br_l
            n = cb * 4 + q_l
            tf = b_loc * N_TABLES + n
            tok = plsc.load_gather(tok_v, [tf >> 7, tf & (G - 1)])
            idx = tok + jnp.minimum(n, N_TABLES - 1) * VOCAB
            idx_v[j, pl.ds(s * 16, 16)] = idx
        return carry

    lax.fori_loop(0, NGPW, idx_body, 0)
    base = wid * SPW

    def chunk_body(c, carry):
        copies = []
        for g in range(SCG):
            copies.append(
                pltpu.async_copy(
                    table_hbm.at[idx_v.at[c * SCG + g]],
                    buf.at[pl.ds(g * G, G), :],
                    sem,
                )
            )
        for cp in copies:
            cp.wait()
        pltpu.sync_copy(buf, out_hbm.at[pl.ds(base + c * CHUNK, CHUNK), :])
        return carry

    lax.fori_loop(0, NCH, chunk_body, 0)


BLK = 2048
LASTK = N_TABLES * EMB - (NCB - 1) * 128


def _mm_body(e_ref, m_ref, c_ref, o_ref):
    acc = c_ref[...] + jnp.zeros((BLK, NL * HID), jnp.float32)
    for cb in range(NCB):
        x = e_ref[:, cb].reshape(BLK, 128)
        if cb == NCB - 1:
            acc += jnp.dot(x[:, :LASTK], m_ref[cb, :LASTK],
                           preferred_element_type=jnp.float32)
        else:
            acc += jnp.dot(x, m_ref[cb],
                           preferred_element_type=jnp.float32)
    acc_t = acc.T
    o_ref[0] = acc_t[:HID]
    o_ref[1] = acc_t[HID:]


_mm = pl.pallas_call(
    _mm_body,
    grid=(B // BLK,),
    in_specs=[
        pl.BlockSpec((BLK // 8, NCB, 8, 128), lambda i: (i, 0, 0, 0)),
        pl.BlockSpec((NCB, 128, NL * HID), lambda i: (0, 0, 0)),
        pl.BlockSpec((1, NL * HID), lambda i: (0, 0)),
    ],
    out_specs=pl.BlockSpec((NL, HID, BLK), lambda i: (0, 0, i)),
    out_shape=jax.ShapeDtypeStruct((NL, HID, B), jnp.float32),
)


def kernel(tokens, tables, W_embed_lin, b_embed_lin, W_final, b_final):
    tokens_flat = tokens.astype(jnp.int32).reshape(R // G, G)

    tables_t = jnp.transpose(tables, (0, 2, 1))
    tail = tables_t[:, :, VOCAB - 256:]
    packed = _sc_repack(tables_t, tail)
    table_flat = packed.reshape(N_TABLES * VOCAB, EMB)

    M = jnp.einsum("ln,nhd->ndlh", W_final, W_embed_lin).reshape(
        N_TABLES * EMB, NL * HID
    )
    Mp = jnp.concatenate(
        [M.reshape(N_TABLES, EMB, NL * HID),
         jnp.zeros((4 * NCB - N_TABLES, EMB, NL * HID), jnp.float32)]
    ).reshape(NCB, 128, NL * HID)
    const = (W_final @ b_embed_lin + b_final[:, None]).reshape(1, NL * HID)

    rows = _sc_gather(tokens_flat, table_flat)
    E4 = rows.reshape(B // 8, NCB, 8, 128)
    out_t = _mm(E4, Mp, const)
    return jnp.transpose(out_t, (0, 2, 1))


# speedup vs baseline: 2.1412x; 1.0074x over previous
import functools

import jax
import jax.numpy as jnp
from jax import lax
from jax.experimental import pallas as pl
from jax.experimental.pallas import tpu as pltpu
from jax.experimental.pallas import tpu_sc as plsc

N_TABLES = 26
VOCAB = 100000
EMB = 32
HID = 64
NL = 2
B = 16384

R = B * N_TABLES
NC, NS = 2, 16
NW = NC * NS

VC = 768
WPT = 130
VTAIL = 256
UA = N_TABLES * WPT
UA_LO = UA // NW
UA_EXTRA = UA - UA_LO * NW
QROWS = VC * EMB // 128
QTAIL = VTAIL * EMB // 128

NCB = (N_TABLES * EMB + 127) // 128
ER = (B // 8) * NCB * 32
RPW = R // NW
SPW = ER // NW
G = 128
NGPW = SPW // G
SCG = 8
CHUNK = G * SCG
NCH = NGPW // SCG
TPG = RPW // G


@functools.partial(
    pl.kernel,
    out_type=jax.ShapeDtypeStruct((N_TABLES * VOCAB * EMB,), jnp.float32),
    mesh=plsc.VectorSubcoreMesh(core_axis_name="c", subcore_axis_name="s"),
    compiler_params=pltpu.CompilerParams(needs_layout_passes=False),
    scratch_types=[
        pltpu.VMEM((EMB, VC), jnp.float32),
        pltpu.VMEM((EMB, VC), jnp.float32),
        pltpu.VMEM((EMB * (VC + 1),), jnp.float32),
        pltpu.VMEM((VC * EMB,), jnp.float32),
        pltpu.VMEM((VC * EMB,), jnp.float32),
        pltpu.SemaphoreType.DMA,
        pltpu.SemaphoreType.DMA,
        pltpu.SemaphoreType.DMA,
        pltpu.SemaphoreType.DMA,
    ],
)
def _sc_repack(tab_hbm, tail_hbm, packed_hbm, buf0, buf1, pb, ob0, ob1,
               rd0, rd1, wr0, wr1):
    wid = lax.axis_index("s") * NC + lax.axis_index("c")
    u0 = wid * UA_LO + jnp.minimum(wid, UA_EXTRA)
    cnt = UA_LO + jnp.where(wid < UA_EXTRA, 1, 0)
    u1 = u0 + cnt

    bufs = (buf0, buf1)
    obs = (ob0, ob1)
    rds = (rd0, rd1)
    wrs = (wr0, wr1)
    lanes = lax.iota(jnp.int32, 16)
    OUTW = VC * EMB

    def unit_nv(u):
        n = u // WPT
        v0 = pl.multiple_of((u % WPT) * VC, 128)
        return n, v0

    PITCH = VC + 1

    def fire_read(u, p):
        n, v0 = unit_nv(u)
        pltpu.async_copy(tab_hbm.at[n, :, pl.ds(v0, VC)], bufs[p], rds[p])

    fire_read(u0, 0)
    fire_read(u0 + 1, 1)

    def transpose_window(buf, ob, nv):
        @plsc.parallel_loop(0, nv // 16, unroll=2)
        def _repitch(s):
            v = s * 16
            for d in range(EMB):
                pb[pl.ds(d * PITCH + v, 16)] = buf[d, pl.ds(v, 16)]

        idx0 = lanes * PITCH
        idx_hi = idx0 + 16 * PITCH

        @plsc.parallel_loop(0, nv, unroll=4,
                            carry=(idx0, idx_hi, jnp.int32(0)))
        def _steps(v, c):
            ia, ib, off = c
            g0 = plsc.load_gather(pb, [ia])
            g1 = plsc.load_gather(pb, [ib])
            ob[pl.ds(off, 16)] = g0
            ob[pl.ds(off + 16, 16)] = g1
            return ia + 1, ib + 1, off + EMB

    def loop_body(i2, carry):
        for p in range(2):
            u = u0 + 2 * i2 + p

            @pl.when(u < u1)
            def _do():
                n, v0 = unit_nv(u)
                pltpu.make_async_copy(
                    tab_hbm.at[0, :, pl.ds(0, VC)], bufs[p], rds[p]
                ).wait()

                @pl.when(u >= u0 + 2)
                def _wait_prev_write():
                    pltpu.make_async_copy(
                        obs[p], packed_hbm.at[pl.ds(0, OUTW)], wrs[p]
                    ).wait()

                transpose_window(bufs[p], obs[p], VC)
                flat0 = pl.multiple_of(n * (VOCAB * EMB) + v0 * EMB, 1024)
                pltpu.async_copy(
                    obs[p], packed_hbm.at[pl.ds(flat0, OUTW)], wrs[p]
                )

                @pl.when(u + 2 < u1)
                def _next_read():
                    fire_read(u + 2, p)

        return carry

    lax.fori_loop(0, (cnt + 1) // 2, loop_body, 0)

    for p in range(2):
        pltpu.make_async_copy(
            obs[p], packed_hbm.at[pl.ds(0, OUTW)], wrs[p]
        ).wait()

    @pl.when(wid < N_TABLES)
    def _tail():
        n = wid
        pltpu.sync_copy(tail_hbm.at[n], buf0.at[:, pl.ds(0, VTAIL)])
        transpose_window(buf0, ob0, VTAIL)
        flat0 = pl.multiple_of(
            n * (VOCAB * EMB) + (VOCAB - VTAIL) * EMB, 1024
        )
        pltpu.sync_copy(ob0.at[pl.ds(0, VTAIL * EMB)],
                        packed_hbm.at[pl.ds(flat0, VTAIL * EMB)])


@functools.partial(
    pl.kernel,
    out_type=jax.ShapeDtypeStruct((ER, EMB), jnp.float32),
    mesh=plsc.VectorSubcoreMesh(core_axis_name="c", subcore_axis_name="s"),
    compiler_params=pltpu.CompilerParams(
        use_tc_tiling_on_sc=False, needs_layout_passes=False
    ),
    scratch_types=[
        pltpu.VMEM((N_TABLES, B // NW), jnp.int32),
        pltpu.VMEM((NGPW, G), jnp.int32),
        pltpu.VMEM((CHUNK, EMB), jnp.float32),
        pltpu.SemaphoreType.DMA,
    ],
)
def _sc_gather(tokens_hbm, table_hbm, out_hbm, tok_v, idx_v, buf, sem):
    wid = lax.axis_index("s") * NC + lax.axis_index("c")

    pltpu.sync_copy(tokens_hbm.at[:, pl.ds(wid * (B // NW), B // NW)], tok_v)

    lanes = lax.iota(jnp.int32, 16)
    br_l = lanes >> 2
    q_l = lanes & 3

    def idx_body(j, carry):
        for s in range(G // 16):
            off = j * G + s * 16
            blk = off >> 5
            half = (off & 31) >> 4
            bb = blk // NCB
            cb = blk % NCB
            b_loc = bb * 8 + half * 4 + br_l
            n = cb * 4 + q_l
            tok = plsc.load_gather(tok_v, [jnp.minimum(n, N_TABLES - 1),
                                           b_loc])
            idx = tok + jnp.minimum(n, N_TABLES - 1) * VOCAB
            idx_v[j, pl.ds(s * 16, 16)] = idx
        return carry

    lax.fori_loop(0, NGPW, idx_body, 0)
    base = wid * SPW

    def chunk_body(c, carry):
        copies = []
        for g in range(SCG):
            copies.append(
                pltpu.async_copy(
                    table_hbm.at[idx_v.at[c * SCG + g]],
                    buf.at[pl.ds(g * G, G), :],
                    sem,
                )
            )
        for cp in copies:
            cp.wait()
        pltpu.sync_copy(buf, out_hbm.at[pl.ds(base + c * CHUNK, CHUNK), :])
        return carry

    lax.fori_loop(0, NCH, chunk_body, 0)


BLK = 2048
LASTK = N_TABLES * EMB - (NCB - 1) * 128


def _mm_body(e_ref, m_ref, c_ref, o_ref):
    acc = c_ref[...] + jnp.zeros((BLK, NL * HID), jnp.float32)
    for cb in range(NCB):
        x = e_ref[:, cb].reshape(BLK, 128)
        if cb == NCB - 1:
            acc += jnp.dot(x[:, :LASTK], m_ref[cb, :LASTK],
                           preferred_element_type=jnp.float32)
        else:
            acc += jnp.dot(x, m_ref[cb],
                           preferred_element_type=jnp.float32)
    acc_t = acc.T
    o_ref[0] = acc_t[:HID]
    o_ref[1] = acc_t[HID:]


_mm = pl.pallas_call(
    _mm_body,
    grid=(B // BLK,),
    in_specs=[
        pl.BlockSpec((BLK // 8, NCB, 8, 128), lambda i: (i, 0, 0, 0)),
        pl.BlockSpec((NCB, 128, NL * HID), lambda i: (0, 0, 0)),
        pl.BlockSpec((1, NL * HID), lambda i: (0, 0)),
    ],
    out_specs=pl.BlockSpec((NL, HID, BLK), lambda i: (0, 0, i)),
    out_shape=jax.ShapeDtypeStruct((NL, HID, B), jnp.float32),
)


def kernel(tokens, tables, W_embed_lin, b_embed_lin, W_final, b_final):
    tokens_t = tokens.astype(jnp.int32).T

    tables_t = jnp.transpose(tables, (0, 2, 1))
    tail = tables_t[:, :, VOCAB - 256:]
    packed = _sc_repack(tables_t, tail)
    table_flat = packed.reshape(N_TABLES * VOCAB, EMB)

    M = jnp.einsum("ln,nhd->ndlh", W_final, W_embed_lin).reshape(
        N_TABLES * EMB, NL * HID
    )
    Mp = jnp.concatenate(
        [M.reshape(N_TABLES, EMB, NL * HID),
         jnp.zeros((4 * NCB - N_TABLES, EMB, NL * HID), jnp.float32)]
    ).reshape(NCB, 128, NL * HID)
    const = (W_final @ b_embed_lin + b_final[:, None]).reshape(1, NL * HID)

    rows = _sc_gather(tokens_t, table_flat)
    E4 = rows.reshape(B // 8, NCB, 8, 128)
    out_t = _mm(E4, Mp, const)
    return jnp.transpose(out_t, (0, 2, 1))


# speedup vs baseline: 2.1946x; 1.0249x over previous
import functools

import jax
import jax.numpy as jnp
from jax import lax
from jax.experimental import pallas as pl
from jax.experimental.pallas import tpu as pltpu
from jax.experimental.pallas import tpu_sc as plsc

N_TABLES = 26
VOCAB = 100000
EMB = 32
HID = 64
NL = 2
B = 16384

R = B * N_TABLES
NC, NS = 2, 16
NW = NC * NS

VC = 768
WPT = 130
VTAIL = 256
UA = N_TABLES * WPT
UA_LO = UA // NW
UA_EXTRA = UA - UA_LO * NW
QROWS = VC * EMB // 128
QTAIL = VTAIL * EMB // 128

NCB = (N_TABLES * EMB + 127) // 128
ER = (B // 8) * NCB * 32
RPW = R // NW
SPW = ER // NW
G = 128
NGPW = SPW // G
SCG = 8
CHUNK = G * SCG
NCH = NGPW // SCG
TPG = RPW // G


@functools.partial(
    pl.kernel,
    out_type=jax.ShapeDtypeStruct((N_TABLES * VOCAB * EMB,), jnp.float32),
    mesh=plsc.VectorSubcoreMesh(core_axis_name="c", subcore_axis_name="s"),
    compiler_params=pltpu.CompilerParams(needs_layout_passes=False),
    scratch_types=[
        pltpu.VMEM((EMB, VC), jnp.float32),
        pltpu.VMEM((EMB, VC), jnp.float32),
        pltpu.VMEM((EMB * (VC + 1),), jnp.float32),
        pltpu.VMEM((VC * EMB,), jnp.float32),
        pltpu.VMEM((VC * EMB,), jnp.float32),
        pltpu.SemaphoreType.DMA,
        pltpu.SemaphoreType.DMA,
        pltpu.SemaphoreType.DMA,
        pltpu.SemaphoreType.DMA,
    ],
)
def _sc_repack(tab_hbm, tail_hbm, packed_hbm, buf0, buf1, pb, ob0, ob1,
               rd0, rd1, wr0, wr1):
    wid = lax.axis_index("s") * NC + lax.axis_index("c")
    u0 = wid * UA_LO + jnp.minimum(wid, UA_EXTRA)
    cnt = UA_LO + jnp.where(wid < UA_EXTRA, 1, 0)
    u1 = u0 + cnt

    bufs = (buf0, buf1)
    obs = (ob0, ob1)
    rds = (rd0, rd1)
    wrs = (wr0, wr1)
    lanes = lax.iota(jnp.int32, 16)
    OUTW = VC * EMB

    def unit_nv(u):
        n = u // WPT
        v0 = pl.multiple_of((u % WPT) * VC, 128)
        return n, v0

    PITCH = VC + 1

    def fire_read(u, p):
        n, v0 = unit_nv(u)
        pltpu.async_copy(tab_hbm.at[n, :, pl.ds(v0, VC)], bufs[p], rds[p])

    fire_read(u0, 0)
    fire_read(u0 + 1, 1)

    def transpose_window(buf, ob, nv):
        @plsc.parallel_loop(0, nv // 16, unroll=2)
        def _repitch(s):
            v = s * 16
            for d in range(EMB):
                pb[pl.ds(d * PITCH + v, 16)] = buf[d, pl.ds(v, 16)]

        idx0 = lanes * PITCH
        idx_hi = idx0 + 16 * PITCH

        @plsc.parallel_loop(0, nv, unroll=4,
                            carry=(idx0, idx_hi, jnp.int32(0)))
        def _steps(v, c):
            ia, ib, off = c
            g0 = plsc.load_gather(pb, [ia])
            g1 = plsc.load_gather(pb, [ib])
            ob[pl.ds(off, 16)] = g0
            ob[pl.ds(off + 16, 16)] = g1
            return ia + 1, ib + 1, off + EMB

    def loop_body(i2, carry):
        for p in range(2):
            u = u0 + 2 * i2 + p

            @pl.when(u < u1)
            def _do():
                n, v0 = unit_nv(u)
                pltpu.make_async_copy(
                    tab_hbm.at[0, :, pl.ds(0, VC)], bufs[p], rds[p]
                ).wait()

                @pl.when(u >= u0 + 2)
                def _wait_prev_write():
                    pltpu.make_async_copy(
                        obs[p], packed_hbm.at[pl.ds(0, OUTW)], wrs[p]
                    ).wait()

                transpose_window(bufs[p], obs[p], VC)
                flat0 = pl.multiple_of(n * (VOCAB * EMB) + v0 * EMB, 1024)
                pltpu.async_copy(
                    obs[p], packed_hbm.at[pl.ds(flat0, OUTW)], wrs[p]
                )

                @pl.when(u + 2 < u1)
                def _next_read():
                    fire_read(u + 2, p)

        return carry

    lax.fori_loop(0, (cnt + 1) // 2, loop_body, 0)

    for p in range(2):
        pltpu.make_async_copy(
            obs[p], packed_hbm.at[pl.ds(0, OUTW)], wrs[p]
        ).wait()

    @pl.when(wid < N_TABLES)
    def _tail():
        n = wid
        pltpu.sync_copy(tail_hbm.at[n], buf0.at[:, pl.ds(0, VTAIL)])
        transpose_window(buf0, ob0, VTAIL)
        flat0 = pl.multiple_of(
            n * (VOCAB * EMB) + (VOCAB - VTAIL) * EMB, 1024
        )
        pltpu.sync_copy(ob0.at[pl.ds(0, VTAIL * EMB)],
                        packed_hbm.at[pl.ds(flat0, VTAIL * EMB)])


@functools.partial(
    pl.kernel,
    out_type=jax.ShapeDtypeStruct((ER, EMB), jnp.float32),
    mesh=plsc.VectorSubcoreMesh(core_axis_name="c", subcore_axis_name="s"),
    compiler_params=pltpu.CompilerParams(
        use_tc_tiling_on_sc=False, needs_layout_passes=False
    ),
    scratch_types=[
        pltpu.VMEM((N_TABLES, B // NW), jnp.int32),
        pltpu.VMEM((NGPW, G), jnp.int32),
        pltpu.VMEM((CHUNK, EMB), jnp.float32),
        pltpu.VMEM((CHUNK, EMB), jnp.float32),
        pltpu.VMEM((CHUNK, EMB), jnp.float32),
        pltpu.SemaphoreType.DMA,
        pltpu.SemaphoreType.DMA,
        pltpu.SemaphoreType.DMA,
        pltpu.SemaphoreType.DMA,
        pltpu.SemaphoreType.DMA,
        pltpu.SemaphoreType.DMA,
    ],
)
def _sc_gather(tokens_hbm, table_hbm, out_hbm, tok_v, idx_v,
               bA, bB, bC, gA, gB, gC, sA, sB, sC):
    wid = lax.axis_index("s") * NC + lax.axis_index("c")

    pltpu.sync_copy(tokens_hbm.at[:, pl.ds(wid * (B // NW), B // NW)], tok_v)

    lanes = lax.iota(jnp.int32, 16)
    br_l = lanes >> 2
    q_l = lanes & 3

    def idx_body(j, carry):
        for s in range(G // 16):
            off = j * G + s * 16
            blk = off >> 5
            half = (off & 31) >> 4
            bb = blk // NCB
            cb = blk % NCB
            b_loc = bb * 8 + half * 4 + br_l
            n = cb * 4 + q_l
            tok = plsc.load_gather(tok_v, [jnp.minimum(n, N_TABLES - 1),
                                           b_loc])
            idx = tok + jnp.minimum(n, N_TABLES - 1) * VOCAB
            idx_v[j, pl.ds(s * 16, 16)] = idx
        return carry

    lax.fori_loop(0, NGPW, idx_body, 0)
    base = wid * SPW

    bufs3 = (bA, bB, bC)
    gsem = (gA, gB, gC)
    ssem = (sA, sB, sC)

    def fire_g(c, p):
        for g in range(SCG):
            pltpu.async_copy(
                table_hbm.at[idx_v.at[c * SCG + g]],
                bufs3[p].at[pl.ds(g * G, G), :],
                gsem[p],
            )

    def wait_g(p):
        for g in range(SCG):
            pltpu.make_async_copy(
                table_hbm.at[idx_v.at[0]],
                bufs3[p].at[pl.ds(g * G, G), :],
                gsem[p],
            ).wait()

    def wait_s(p):
        pltpu.make_async_copy(
            bufs3[p], out_hbm.at[pl.ds(0, CHUNK), :], ssem[p]
        ).wait()

    fire_g(0, 0)
    fire_g(1, 1)

    def chunk_body(i4, carry):
        for p in range(3):
            c = 3 * i4 + p

            @pl.when(c < NCH)
            def _step():
                wait_g(p)
                pltpu.async_copy(
                    bufs3[p],
                    out_hbm.at[pl.ds(base + c * CHUNK, CHUNK), :],
                    ssem[p],
                )

                @pl.when(c >= 1)
                def _wait_prev_store():
                    wait_s((p + 2) % 3)

                @pl.when(c + 2 < NCH)
                def _next_gathers():
                    fire_g(c + 2, (p + 2) % 3)

        return carry

    lax.fori_loop(0, (NCH + 2) // 3, chunk_body, 0)
    wait_s((NCH - 1) % 3)


BLK = 2048
LASTK = N_TABLES * EMB - (NCB - 1) * 128


def _mm_body(e_ref, m_ref, c_ref, o_ref):
    acc = c_ref[...] + jnp.zeros((BLK, NL * HID), jnp.float32)
    for cb in range(NCB):
        x = e_ref[:, cb].reshape(BLK, 128)
        if cb == NCB - 1:
            acc += jnp.dot(x[:, :LASTK], m_ref[cb, :LASTK],
                           preferred_element_type=jnp.float32)
        else:
            acc += jnp.dot(x, m_ref[cb],
                           preferred_element_type=jnp.float32)
    acc_t = acc.T
    o_ref[0] = acc_t[:HID]
    o_ref[1] = acc_t[HID:]


_mm = pl.pallas_call(
    _mm_body,
    grid=(B // BLK,),
    in_specs=[
        pl.BlockSpec((BLK // 8, NCB, 8, 128), lambda i: (i, 0, 0, 0)),
        pl.BlockSpec((NCB, 128, NL * HID), lambda i: (0, 0, 0)),
        pl.BlockSpec((1, NL * HID), lambda i: (0, 0)),
    ],
    out_specs=pl.BlockSpec((NL, HID, BLK), lambda i: (0, 0, i)),
    out_shape=jax.ShapeDtypeStruct((NL, HID, B), jnp.float32),
)


def kernel(tokens, tables, W_embed_lin, b_embed_lin, W_final, b_final):
    tokens_t = tokens.astype(jnp.int32).T

    tables_t = jnp.transpose(tables, (0, 2, 1))
    tail = tables_t[:, :, VOCAB - 256:]
    packed = _sc_repack(tables_t, tail)
    table_flat = packed.reshape(N_TABLES * VOCAB, EMB)

    M = jnp.einsum("ln,nhd->ndlh", W_final, W_embed_lin).reshape(
        N_TABLES * EMB, NL * HID
    )
    Mp = jnp.concatenate(
        [M.reshape(N_TABLES, EMB, NL * HID),
         jnp.zeros((4 * NCB - N_TABLES, EMB, NL * HID), jnp.float32)]
    ).reshape(NCB, 128, NL * HID)
    const = (W_final @ b_embed_lin + b_final[:, None]).reshape(1, NL * HID)

    rows = _sc_gather(tokens_t, table_flat)
    E4 = rows.reshape(B // 8, NCB, 8, 128)
    out_t = _mm(E4, Mp, const)
    return jnp.transpose(out_t, (0, 2, 1))


# speedup vs baseline: 2.1948x; 1.0001x over previous
import functools

import jax
import jax.numpy as jnp
from jax import lax
from jax.experimental import pallas as pl
from jax.experimental.pallas import tpu as pltpu
from jax.experimental.pallas import tpu_sc as plsc

N_TABLES = 26
VOCAB = 100000
EMB = 32
HID = 64
NL = 2
B = 16384

R = B * N_TABLES
NC, NS = 2, 16
NW = NC * NS

VC = 768
WPT = 130
VTAIL = 256
UA = N_TABLES * WPT
UA_LO = UA // NW
UA_EXTRA = UA - UA_LO * NW
QROWS = VC * EMB // 128
QTAIL = VTAIL * EMB // 128

NCB = (N_TABLES * EMB + 127) // 128
ER = (B // 8) * NCB * 32
RPW = R // NW
SPW = ER // NW
G = 128
NGPW = SPW // G
SCG = 8
CHUNK = G * SCG
NCH = NGPW // SCG


@functools.partial(
    pl.kernel,
    out_type=jax.ShapeDtypeStruct((N_TABLES * VOCAB * EMB,), jnp.float32),
    mesh=plsc.VectorSubcoreMesh(core_axis_name="c", subcore_axis_name="s"),
    compiler_params=pltpu.CompilerParams(needs_layout_passes=False),
    scratch_types=[
        pltpu.VMEM((EMB, VC), jnp.float32),
        pltpu.VMEM((EMB, VC), jnp.float32),
        pltpu.VMEM((EMB * (VC + 1),), jnp.float32),
        pltpu.VMEM((VC * EMB,), jnp.float32),
        pltpu.VMEM((VC * EMB,), jnp.float32),
        pltpu.SemaphoreType.DMA,
        pltpu.SemaphoreType.DMA,
        pltpu.SemaphoreType.DMA,
        pltpu.SemaphoreType.DMA,
    ],
)
def _sc_repack(tab_hbm, tail_hbm, packed_hbm, buf0, buf1, pb, ob0, ob1,
               rd0, rd1, wr0, wr1):
    wid = lax.axis_index("s") * NC + lax.axis_index("c")
    u0 = wid * UA_LO + jnp.minimum(wid, UA_EXTRA)
    cnt = UA_LO + jnp.where(wid < UA_EXTRA, 1, 0)
    u1 = u0 + cnt

    bufs = (buf0, buf1)
    obs = (ob0, ob1)
    rds = (rd0, rd1)
    wrs = (wr0, wr1)
    lanes = lax.iota(jnp.int32, 16)
    OUTW = VC * EMB

    def unit_nv(u):
        n = u // WPT
        v0 = pl.multiple_of((u % WPT) * VC, 128)
        return n, v0

    PITCH = VC + 1

    def fire_read(u, p):
        n, v0 = unit_nv(u)
        pltpu.async_copy(tab_hbm.at[n, :, pl.ds(v0, VC)], bufs[p], rds[p])

    fire_read(u0, 0)
    fire_read(u0 + 1, 1)

    def transpose_window(buf, ob, nv):
        @plsc.parallel_loop(0, nv // 16, unroll=2)
        def _repitch(s):
            v = s * 16
            for d in range(EMB):
                pb[pl.ds(d * PITCH + v, 16)] = buf[d, pl.ds(v, 16)]

        idx0 = lanes * PITCH
        idx_hi = idx0 + 16 * PITCH

        @plsc.parallel_loop(0, nv, unroll=4,
                            carry=(idx0, idx_hi, jnp.int32(0)))
        def _steps(v, c):
            ia, ib, off = c
            g0 = plsc.load_gather(pb, [ia])
            g1 = plsc.load_gather(pb, [ib])
            ob[pl.ds(off, 16)] = g0
            ob[pl.ds(off + 16, 16)] = g1
            return ia + 1, ib + 1, off + EMB

    def loop_body(i2, carry):
        for p in range(2):
            u = u0 + 2 * i2 + p

            @pl.when(u < u1)
            def _do():
                n, v0 = unit_nv(u)
                pltpu.make_async_copy(
                    tab_hbm.at[0, :, pl.ds(0, VC)], bufs[p], rds[p]
                ).wait()

                @pl.when(u >= u0 + 2)
                def _wait_prev_write():
                    pltpu.make_async_copy(
                        obs[p], packed_hbm.at[pl.ds(0, OUTW)], wrs[p]
                    ).wait()

                transpose_window(bufs[p], obs[p], VC)
                flat0 = pl.multiple_of(n * (VOCAB * EMB) + v0 * EMB, 1024)
                pltpu.async_copy(
                    obs[p], packed_hbm.at[pl.ds(flat0, OUTW)], wrs[p]
                )

                @pl.when(u + 2 < u1)
                def _next_read():
                    fire_read(u + 2, p)

        return carry

    lax.fori_loop(0, (cnt + 1) // 2, loop_body, 0)

    for p in range(2):
        pltpu.make_async_copy(
            obs[p], packed_hbm.at[pl.ds(0, OUTW)], wrs[p]
        ).wait()

    @pl.when(wid < N_TABLES)
    def _tail():
        n = wid
        pltpu.sync_copy(tail_hbm.at[n], buf0.at[:, pl.ds(0, VTAIL)])
        transpose_window(buf0, ob0, VTAIL)
        flat0 = pl.multiple_of(
            n * (VOCAB * EMB) + (VOCAB - VTAIL) * EMB, 1024
        )
        pltpu.sync_copy(ob0.at[pl.ds(0, VTAIL * EMB)],
                        packed_hbm.at[pl.ds(flat0, VTAIL * EMB)])


@functools.partial(
    pl.kernel,
    out_type=jax.ShapeDtypeStruct((ER, EMB), jnp.float32),
    mesh=plsc.VectorSubcoreMesh(core_axis_name="c", subcore_axis_name="s"),
    compiler_params=pltpu.CompilerParams(
        use_tc_tiling_on_sc=False, needs_layout_passes=False
    ),
    scratch_types=[
        pltpu.VMEM((N_TABLES, B // NW), jnp.int32),
        pltpu.VMEM((NGPW, G), jnp.int32),
        pltpu.VMEM((CHUNK, EMB), jnp.float32),
        pltpu.VMEM((CHUNK, EMB), jnp.float32),
        pltpu.VMEM((CHUNK, EMB), jnp.float32),
        pltpu.SemaphoreType.DMA,
        pltpu.SemaphoreType.DMA,
        pltpu.SemaphoreType.DMA,
        pltpu.SemaphoreType.DMA,
        pltpu.SemaphoreType.DMA,
        pltpu.SemaphoreType.DMA,
    ],
)
def _sc_gather(tokens_hbm, table_hbm, out_hbm, tok_v, idx_v,
               bA, bB, bC, gA, gB, gC, sA, sB, sC):
    wid = lax.axis_index("s") * NC + lax.axis_index("c")

    pltpu.sync_copy(tokens_hbm.at[:, pl.ds(wid * (B // NW), B // NW)], tok_v)

    lanes = lax.iota(jnp.int32, 16)
    br_l = lanes >> 2
    q_l = lanes & 3

    def idx_body(j, carry):
        for s in range(G // 16):
            off = j * G + s * 16
            blk = off >> 5
            half = (off & 31) >> 4
            bb = blk // NCB
            cb = blk % NCB
            b_loc = bb * 8 + half * 4 + br_l
            n = cb * 4 + q_l
            tok = plsc.load_gather(tok_v, [jnp.minimum(n, N_TABLES - 1),
                                           b_loc])
            idx = tok + jnp.minimum(n, N_TABLES - 1) * VOCAB
            idx_v[j, pl.ds(s * 16, 16)] = idx
        return carry

    lax.fori_loop(0, NGPW, idx_body, 0)
    base = wid * SPW

    bufs3 = (bA, bB, bC)
    gsem = (gA, gB, gC)
    ssem = (sA, sB, sC)

    def fire_g(c, p):
        for g in range(SCG):
            pltpu.async_copy(
                table_hbm.at[idx_v.at[c * SCG + g]],
                bufs3[p].at[pl.ds(g * G, G), :],
                gsem[p],
            )

    def wait_g(p):
        for g in range(SCG):
            pltpu.make_async_copy(
                table_hbm.at[idx_v.at[0]],
                bufs3[p].at[pl.ds(g * G, G), :],
                gsem[p],
            ).wait()

    def wait_s(p):
        pltpu.make_async_copy(
            bufs3[p], out_hbm.at[pl.ds(0, CHUNK), :], ssem[p]
        ).wait()

    fire_g(0, 0)
    fire_g(1, 1)

    def chunk_body(i4, carry):
        for p in range(3):
            c = 3 * i4 + p

            @pl.when(c < NCH)
            def _step():
                wait_g(p)
                pltpu.async_copy(
                    bufs3[p],
                    out_hbm.at[pl.ds(base + c * CHUNK, CHUNK), :],
                    ssem[p],
                )

                @pl.when(c >= 1)
                def _wait_prev_store():
                    wait_s((p + 2) % 3)

                @pl.when(c + 2 < NCH)
                def _next_gathers():
                    fire_g(c + 2, (p + 2) % 3)

        return carry

    lax.fori_loop(0, (NCH + 2) // 3, chunk_body, 0)
    wait_s((NCH - 1) % 3)


BLK = 2048
LASTK = N_TABLES * EMB - (NCB - 1) * 128


def _mm_body(e_ref, m_ref, c_ref, o_ref):
    acc = c_ref[...] + jnp.zeros((BLK, NL * HID), jnp.float32)
    for cb in range(NCB):
        x = e_ref[:, cb].reshape(BLK, 128)
        if cb == NCB - 1:
            acc += jnp.dot(x[:, :LASTK], m_ref[cb, :LASTK],
                           preferred_element_type=jnp.float32)
        else:
            acc += jnp.dot(x, m_ref[cb],
                           preferred_element_type=jnp.float32)
    acc_t = acc.T
    o_ref[0] = acc_t[:HID]
    o_ref[1] = acc_t[HID:]


_mm = pl.pallas_call(
    _mm_body,
    grid=(B // BLK,),
    in_specs=[
        pl.BlockSpec((BLK // 8, NCB, 8, 128), lambda i: (i, 0, 0, 0)),
        pl.BlockSpec((NCB, 128, NL * HID), lambda i: (0, 0, 0)),
        pl.BlockSpec((1, NL * HID), lambda i: (0, 0)),
    ],
    out_specs=pl.BlockSpec((NL, HID, BLK), lambda i: (0, 0, i)),
    out_shape=jax.ShapeDtypeStruct((NL, HID, B), jnp.float32),
)


def kernel(tokens, tables, W_embed_lin, b_embed_lin, W_final, b_final):
    tokens_t = tokens.astype(jnp.int32).T

    tables_t = jnp.transpose(tables, (0, 2, 1))
    tail = tables_t[:, :, VOCAB - 256:]
    packed = _sc_repack(tables_t, tail)
    table_flat = packed.reshape(N_TABLES * VOCAB, EMB)

    M = jnp.einsum("ln,nhd->ndlh", W_final, W_embed_lin).reshape(
        N_TABLES * EMB, NL * HID
    )
    Mp = jnp.concatenate(
        [M.reshape(N_TABLES, EMB, NL * HID),
         jnp.zeros((4 * NCB - N_TABLES, EMB, NL * HID), jnp.float32)]
    ).reshape(NCB, 128, NL * HID)
    const = (W_final @ b_embed_lin + b_final[:, None]).reshape(1, NL * HID)

    rows = _sc_gather(tokens_t, table_flat)
    E4 = rows.reshape(B // 8, NCB, 8, 128)
    out_t = _mm(E4, Mp, const)
    return jnp.transpose(out_t, (0, 2, 1))
